# baseline (speedup 1.0000x reference)
import functools
import math

import jax
import jax.numpy as jnp
from jax import lax
from jax.experimental import pallas as pl
from jax.experimental.pallas import tpu as pltpu

N = 2
LQ = 4096
D_MODEL = 256
D_HEAD = 64
N_HEADS = 8
N_LEVELS = 2
N_POINTS = 4
H0, W0 = 128, 128
H1, W1 = 64, 64
LS0, LS1 = 0, H0 * W0
LEN_IN = H0 * W0 + H1 * W1


def _vproj_body(x_ref, w_ref, b_ref, o_ref):
    x = x_ref[...]
    w = w_ref[...]
    o_ref[...] = lax.dot_general(
        x, w, (((1,), (1,)), ((), ())), preferred_element_type=jnp.float32
    ) + b_ref[...]


def _value_projection(x, w_v, b_v):
    rows = x.shape[0]
    bl = 2048
    grid = (rows // bl,)
    return pl.pallas_call(
        _vproj_body,
        grid=grid,
        in_specs=[
            pl.BlockSpec((bl, D_MODEL), lambda i: (i, 0)),
            pl.BlockSpec((N_HEADS * D_HEAD, D_MODEL), lambda i: (0, 0)),
            pl.BlockSpec((1, N_HEADS * D_HEAD), lambda i: (0, 0)),
        ],
        out_specs=pl.BlockSpec((bl, N_HEADS * D_HEAD), lambda i: (i, 0)),
        out_shape=jax.ShapeDtypeStruct((rows, N_HEADS * D_HEAD), jnp.float32),
    )(x, w_v, b_v.reshape(1, -1))


def _soaw_body(q_ref, wso_ref, bso_ref, waw_ref, baw_ref, so_ref, aw_ref):
    q = q_ref[...]
    so = lax.dot_general(
        q, wso_ref[...], (((1,), (1,)), ((), ())), preferred_element_type=jnp.float32
    ) + bso_ref[...]
    so_ref[...] = so
    logits = lax.dot_general(
        q, waw_ref[...], (((1,), (1,)), ((), ())), preferred_element_type=jnp.float32
    ) + baw_ref[...]
    m = jnp.max(logits, axis=-1, keepdims=True)
    e = jnp.exp(logits - m)
    r = lax.broadcasted_iota(jnp.int32, (64, 64), 0) // 8
    c = lax.broadcasted_iota(jnp.int32, (64, 64), 1) // 8
    g = (r == c).astype(jnp.float32)
    denom = lax.dot_general(
        e, g, (((1,), (0,)), ((), ())), preferred_element_type=jnp.float32
    )
    aw_ref[...] = e / denom


def _so_aw(q, w_so, b_so, w_aw, b_aw):
    rows = q.shape[0]
    bl = 2048
    grid = (rows // bl,)
    return pl.pallas_call(
        _soaw_body,
        grid=grid,
        in_specs=[
            pl.BlockSpec((bl, D_MODEL), lambda i: (i, 0)),
            pl.BlockSpec((128, D_MODEL), lambda i: (0, 0)),
            pl.BlockSpec((1, 128), lambda i: (0, 0)),
            pl.BlockSpec((64, D_MODEL), lambda i: (0, 0)),
            pl.BlockSpec((1, 64), lambda i: (0, 0)),
        ],
        out_specs=[
            pl.BlockSpec((bl, 128), lambda i: (i, 0)),
            pl.BlockSpec((bl, 64), lambda i: (i, 0)),
        ],
        out_shape=[
            jax.ShapeDtypeStruct((rows, 128), jnp.float32),
            jax.ShapeDtypeStruct((rows, 64), jnp.float32),
        ],
    )(q, w_so, b_so.reshape(1, -1), w_aw, b_aw.reshape(1, -1))


def _oproj_body(x_ref, w_ref, b_ref, o_ref):
    o_ref[...] = lax.dot_general(
        x_ref[...], w_ref[...], (((1,), (1,)), ((), ())),
        preferred_element_type=jnp.float32,
    ) + b_ref[...]


def _out_projection(x, w_o, b_o):
    rows = x.shape[0]
    bl = 2048
    grid = (rows // bl,)
    return pl.pallas_call(
        _oproj_body,
        grid=grid,
        in_specs=[
            pl.BlockSpec((bl, N_HEADS * D_HEAD), lambda i: (i, 0)),
            pl.BlockSpec((D_MODEL, N_HEADS * D_HEAD), lambda i: (0, 0)),
            pl.BlockSpec((1, D_MODEL), lambda i: (0, 0)),
        ],
        out_specs=pl.BlockSpec((bl, D_MODEL), lambda i: (i, 0)),
        out_shape=jax.ShapeDtypeStruct((rows, D_MODEL), jnp.float32),
    )(x, w_o, b_o.reshape(1, -1))


def _indices_weights(reference_points, so, aw):
    idx_parts = []
    w_parts = []
    for l, (H_, W_, ls) in enumerate(((H0, W0, LS0), (H1, W1, LS1))):
        ref = reference_points[:, :, None, l, :]
        sx = so[:, :, :, l, :, 0]
        sy = so[:, :, :, l, :, 1]
        wf = float(W_)
        hf = float(H_)
        lx = (ref[..., 0:1] / wf + sx / wf) * wf - 0.5
        ly = (ref[..., 1:2] / hf + sy / hf) * hf - 0.5
        x0 = jnp.floor(lx)
        y0 = jnp.floor(ly)
        wx1 = lx - x0
        wy1 = ly - y0
        a_l = aw[:, :, :, l, :]
        corner_idx = []
        corner_w = []
        for xi, yi, wgt in (
            (x0, y0, (1 - wx1) * (1 - wy1)),
            (x0 + 1, y0, wx1 * (1 - wy1)),
            (x0, y0 + 1, (1 - wx1) * wy1),
            (x0 + 1, y0 + 1, wx1 * wy1),
        ):
            valid = ((xi >= 0) & (xi < wf) & (yi >= 0) & (yi < hf)).astype(jnp.float32)
            xc = jnp.clip(xi, 0, W_ - 1).astype(jnp.int32)
            yc = jnp.clip(yi, 0, H_ - 1).astype(jnp.int32)
            corner_idx.append(yc * W_ + xc + ls)
            corner_w.append(wgt * valid * a_l)
        idx_parts.append(jnp.stack(corner_idx, axis=-1))
        w_parts.append(jnp.stack(corner_w, axis=-1))
    idx = jnp.stack(idx_parts, axis=3)
    w = jnp.stack(w_parts, axis=3)
    n_ix = lax.broadcasted_iota(jnp.int32, idx.shape, 0)
    h_ix = lax.broadcasted_iota(jnp.int32, idx.shape, 2)
    rows = (n_ix * LEN_IN + idx) * N_HEADS + h_ix
    return (rows.reshape(N * LQ * N_HEADS, 32),
            w.reshape(N * LQ * N_HEADS, 32))


def kernel(query, reference_points, input_flatten, input_spatial_shapes,
           input_level_start_index, W_so, b_so, W_aw, b_aw, W_v, b_v, W_o, b_o):
    value = _value_projection(
        input_flatten.reshape(N * LEN_IN, D_MODEL), W_v, b_v
    )
    so, aw = _so_aw(query.reshape(N * LQ, D_MODEL), W_so, b_so, W_aw, b_aw)
    so = so.reshape(N, LQ, N_HEADS, N_LEVELS, N_POINTS, 2)
    aw = aw.reshape(N, LQ, N_HEADS, N_LEVELS, N_POINTS)
    idx, w = _indices_weights(reference_points, so, aw)

    value_rows = value.reshape(N * LEN_IN * N_HEADS, D_HEAD)
    g = jnp.take(value_rows, idx, axis=0)
    out_rows = jnp.einsum('tc,tcd->td', w, g)

    out = _out_projection(out_rows.reshape(N * LQ, N_HEADS * D_HEAD), W_o, b_o)
    return out.reshape(N, LQ, D_MODEL)


# speedup vs baseline: 9.9607x; 9.9607x over previous
import functools
import math

import jax
import jax.numpy as jnp
from jax import lax
from jax.experimental import pallas as pl
from jax.experimental.pallas import tpu as pltpu
from jax.experimental.pallas import tpu_sc as plsc

N = 2
LQ = 4096
D_MODEL = 256
D_HEAD = 64
N_HEADS = 8
N_LEVELS = 2
N_POINTS = 4
H0, W0 = 128, 128
H1, W1 = 64, 64
LS0, LS1 = 0, H0 * W0
LEN_IN = H0 * W0 + H1 * W1


def _vproj_body(x_ref, w_ref, b_ref, o_ref):
    x = x_ref[...]
    w = w_ref[...]
    o_ref[...] = lax.dot_general(
        x, w, (((1,), (1,)), ((), ())), preferred_element_type=jnp.float32
    ) + b_ref[...]


def _value_projection(x, w_v, b_v):
    rows = x.shape[0]
    bl = 2048
    grid = (rows // bl,)
    return pl.pallas_call(
        _vproj_body,
        grid=grid,
        in_specs=[
            pl.BlockSpec((bl, D_MODEL), lambda i: (i, 0)),
            pl.BlockSpec((N_HEADS * D_HEAD, D_MODEL), lambda i: (0, 0)),
            pl.BlockSpec((1, N_HEADS * D_HEAD), lambda i: (0, 0)),
        ],
        out_specs=pl.BlockSpec((bl, N_HEADS * D_HEAD), lambda i: (i, 0)),
        out_shape=jax.ShapeDtypeStruct((rows, N_HEADS * D_HEAD), jnp.float32),
    )(x, w_v, b_v.reshape(1, -1))


def _soaw_body(q_ref, wso_ref, bso_ref, waw_ref, baw_ref, so_ref, aw_ref):
    q = q_ref[...]
    so = lax.dot_general(
        q, wso_ref[...], (((1,), (1,)), ((), ())), preferred_element_type=jnp.float32
    ) + bso_ref[...]
    so_ref[...] = so
    logits = lax.dot_general(
        q, waw_ref[...], (((1,), (1,)), ((), ())), preferred_element_type=jnp.float32
    ) + baw_ref[...]
    m = jnp.max(logits, axis=-1, keepdims=True)
    e = jnp.exp(logits - m)
    r = lax.broadcasted_iota(jnp.int32, (64, 64), 0) // 8
    c = lax.broadcasted_iota(jnp.int32, (64, 64), 1) // 8
    g = (r == c).astype(jnp.float32)
    denom = lax.dot_general(
        e, g, (((1,), (0,)), ((), ())), preferred_element_type=jnp.float32
    )
    aw_ref[...] = e / denom


def _so_aw(q, w_so, b_so, w_aw, b_aw):
    rows = q.shape[0]
    bl = 2048
    grid = (rows // bl,)
    return pl.pallas_call(
        _soaw_body,
        grid=grid,
        in_specs=[
            pl.BlockSpec((bl, D_MODEL), lambda i: (i, 0)),
            pl.BlockSpec((128, D_MODEL), lambda i: (0, 0)),
            pl.BlockSpec((1, 128), lambda i: (0, 0)),
            pl.BlockSpec((64, D_MODEL), lambda i: (0, 0)),
            pl.BlockSpec((1, 64), lambda i: (0, 0)),
        ],
        out_specs=[
            pl.BlockSpec((bl, 128), lambda i: (i, 0)),
            pl.BlockSpec((bl, 64), lambda i: (i, 0)),
        ],
        out_shape=[
            jax.ShapeDtypeStruct((rows, 128), jnp.float32),
            jax.ShapeDtypeStruct((rows, 64), jnp.float32),
        ],
    )(q, w_so, b_so.reshape(1, -1), w_aw, b_aw.reshape(1, -1))


def _oproj_body(x_ref, w_ref, b_ref, o_ref):
    o_ref[...] = lax.dot_general(
        x_ref[...], w_ref[...], (((1,), (1,)), ((), ())),
        preferred_element_type=jnp.float32,
    ) + b_ref[...]


def _out_projection(x, w_o, b_o):
    rows = x.shape[0]
    bl = 2048
    grid = (rows // bl,)
    return pl.pallas_call(
        _oproj_body,
        grid=grid,
        in_specs=[
            pl.BlockSpec((bl, N_HEADS * D_HEAD), lambda i: (i, 0)),
            pl.BlockSpec((D_MODEL, N_HEADS * D_HEAD), lambda i: (0, 0)),
            pl.BlockSpec((1, D_MODEL), lambda i: (0, 0)),
        ],
        out_specs=pl.BlockSpec((bl, D_MODEL), lambda i: (i, 0)),
        out_shape=jax.ShapeDtypeStruct((rows, D_MODEL), jnp.float32),
    )(x, w_o, b_o.reshape(1, -1))


def _indices_weights(reference_points, so, aw):
    idx_parts = []
    w_parts = []
    for l, (H_, W_, ls) in enumerate(((H0, W0, LS0), (H1, W1, LS1))):
        ref = reference_points[:, :, None, l, :]
        sx = so[:, :, :, l, :, 0]
        sy = so[:, :, :, l, :, 1]
        wf = float(W_)
        hf = float(H_)
        lx = (ref[..., 0:1] / wf + sx / wf) * wf - 0.5
        ly = (ref[..., 1:2] / hf + sy / hf) * hf - 0.5
        x0 = jnp.floor(lx)
        y0 = jnp.floor(ly)
        wx1 = lx - x0
        wy1 = ly - y0
        a_l = aw[:, :, :, l, :]
        corner_idx = []
        corner_w = []
        for xi, yi, wgt in (
            (x0, y0, (1 - wx1) * (1 - wy1)),
            (x0 + 1, y0, wx1 * (1 - wy1)),
            (x0, y0 + 1, (1 - wx1) * wy1),
            (x0 + 1, y0 + 1, wx1 * wy1),
        ):
            valid = ((xi >= 0) & (xi < wf) & (yi >= 0) & (yi < hf)).astype(jnp.float32)
            xc = jnp.clip(xi, 0, W_ - 1).astype(jnp.int32)
            yc = jnp.clip(yi, 0, H_ - 1).astype(jnp.int32)
            corner_idx.append(yc * W_ + xc + ls)
            corner_w.append(wgt * valid * a_l)
        idx_parts.append(jnp.stack(corner_idx, axis=-1))
        w_parts.append(jnp.stack(corner_w, axis=-1))
    idx = jnp.stack(idx_parts, axis=3)
    w = jnp.stack(w_parts, axis=3)
    n_ix = lax.broadcasted_iota(jnp.int32, idx.shape, 0)
    h_ix = lax.broadcasted_iota(jnp.int32, idx.shape, 2)
    rows = (n_ix * LEN_IN + idx) * N_HEADS + h_ix
    return (rows.reshape(N * LQ * N_HEADS, 32),
            w.reshape(N * LQ * N_HEADS, 32))


TOT_ROWS = N * LQ * N_HEADS
N_WORKERS = 32
ROWS_PER_WORKER = TOT_ROWS // N_WORKERS
CHUNK_ROWS = 32
CHUNKS_PER_WORKER = ROWS_PER_WORKER // CHUNK_ROWS
IDX_PER_CHUNK = CHUNK_ROWS * 32
N_STREAMS = IDX_PER_CHUNK // 128


def _splat16(j):
    return lax.full((16,), jnp.int32(j), jnp.int32)


def _sc_gather_body(idx_hbm, w_hbm, value_hbm, out_hbm, idx_v, w_v, g_v,
                    out_v, sem):
    wid = lax.axis_index("s") * 2 + lax.axis_index("c")

    def chunk_body(c, carry):
        g = wid * CHUNKS_PER_WORKER + c
        pltpu.sync_copy(idx_hbm.at[g], idx_v)
        pltpu.sync_copy(w_hbm.at[g], w_v)
        copies = []
        for k in range(N_STREAMS):
            copies.append(pltpu.async_copy(
                value_hbm.at[idx_v.at[pl.ds(k * 128, 128)]],
                g_v.at[pl.ds(k * 128, 128)], sem))
        for cp in copies:
            cp.wait()

        def row_body(r, carry2):
            base = r * 32
            wv = (w_v[pl.ds(base, 16)], w_v[pl.ds(base + 16, 16)])
            accs = [jnp.zeros((16,), jnp.float32) for _ in range(4)]
            for j in range(32):
                wj = jnp.take(wv[j // 16], _splat16(j % 16))
                for c4 in range(4):
                    accs[c4] = accs[c4] + wj * g_v[base + j, pl.ds(c4 * 16, 16)]
            for c4 in range(4):
                out_v[r, pl.ds(c4 * 16, 16)] = accs[c4]
            return carry2

        lax.fori_loop(0, CHUNK_ROWS, row_body, 0)
        pltpu.sync_copy(out_v, out_hbm.at[pl.ds(g * CHUNK_ROWS, CHUNK_ROWS)])
        return carry

    lax.fori_loop(0, CHUNKS_PER_WORKER, chunk_body, 0)


@functools.partial(jax.jit, static_argnums=())
def _sc_gather(idx, w, value_rows):
    run = pl.kernel(
        _sc_gather_body,
        mesh=plsc.VectorSubcoreMesh(core_axis_name="c", subcore_axis_name="s"),
        compiler_params=pltpu.CompilerParams(use_tc_tiling_on_sc=False),
        out_type=jax.ShapeDtypeStruct((TOT_ROWS, D_HEAD), jnp.float32),
        scratch_types=[
            pltpu.VMEM((IDX_PER_CHUNK,), jnp.int32),
            pltpu.VMEM((IDX_PER_CHUNK,), jnp.float32),
            pltpu.VMEM((IDX_PER_CHUNK, D_HEAD), jnp.float32),
            pltpu.VMEM((CHUNK_ROWS, D_HEAD), jnp.float32),
            pltpu.SemaphoreType.DMA,
        ],
    )
    return run(idx.reshape(TOT_ROWS * 32 // IDX_PER_CHUNK, IDX_PER_CHUNK),
               w.reshape(TOT_ROWS * 32 // IDX_PER_CHUNK, IDX_PER_CHUNK),
               value_rows)


def kernel(query, reference_points, input_flatten, input_spatial_shapes,
           input_level_start_index, W_so, b_so, W_aw, b_aw, W_v, b_v, W_o, b_o):
    value = _value_projection(
        input_flatten.reshape(N * LEN_IN, D_MODEL), W_v, b_v
    )
    so, aw = _so_aw(query.reshape(N * LQ, D_MODEL), W_so, b_so, W_aw, b_aw)
    so = so.reshape(N, LQ, N_HEADS, N_LEVELS, N_POINTS, 2)
    aw = aw.reshape(N, LQ, N_HEADS, N_LEVELS, N_POINTS)
    idx, w = _indices_weights(reference_points, so, aw)

    value_rows = value.reshape(N * LEN_IN * N_HEADS, D_HEAD)
    out_rows = _sc_gather(idx, w, value_rows)

    out = _out_projection(out_rows.reshape(N * LQ, N_HEADS * D_HEAD), W_o, b_o)
    return out.reshape(N, LQ, D_MODEL)
